# baseline (speedup 1.0000x reference)
import functools

import jax
import jax.numpy as jnp
from jax import lax
from jax.experimental import pallas as pl
from jax.experimental.pallas import tpu as pltpu
from jax.experimental.pallas import tpu_sc as plsc

L = 64
M = 64
BATCH = 4096
NUP = 33
KK = NUP * NUP
TROWS = 2 * L * NUP
S_STRIDE = L * NUP
I_STRIDE = NUP

_NC, _NS = 2, 16
NW = _NC * _NS
SPW = BATCH // NW
GRP = SPW // 16
PAIRW = 2 * L
NPAIR = SPW // 2


def _sc_body(table_hbm, inputs_hbm, out_hbm, in_v, idx_v, rows0, rows1,
             tmp_v, out_v, sem0, sem1):
    sub = lax.axis_index("s")
    core = lax.axis_index("c")
    wid = sub * _NC + core
    iota = lax.iota(jnp.int32, 16)

    pltpu.sync_copy(inputs_hbm.at[pl.ds(wid * SPW, SPW)], in_v)

    def bidx(t, _):
        carry = jnp.float32(0)
        for k in range(4):
            sv = in_v[t, pl.ds(16 * k, 16)]
            sf = sv.astype(jnp.float32)
            incl = jnp.cumsum(sf)
            nu = (incl - sf + carry).astype(jnp.int32)
            carry = carry + jnp.sum(sf)
            idx_v[pl.ds(t * L + 16 * k, 16)] = (
                sv * S_STRIDE + (iota + 16 * k) * I_STRIDE + nu)
        return 0

    lax.fori_loop(0, SPW, bidx, 0)

    def product(rows_v, off):
        def prod(j, accs):
            accs = list(accs)
            for r in range(8):
                row = off + 8 * j + r
                c = (r % 2) * 4
                for k in range(4):
                    accs[c + k] = accs[c + k] * rows_v[row, pl.ds(16 * k, 16)]
            return tuple(accs)

        ones = jnp.ones((16,), jnp.float32)
        accs = lax.fori_loop(0, L // 8, prod, (ones,) * 8)
        return (accs[0] * accs[4] + accs[1] * accs[5]
                + accs[2] * accs[6] + accs[3] * accs[7])

    def gather_pair(p, dst, sem):
        return pltpu.async_copy(
            table_hbm.at[idx_v.at[pl.ds(p * PAIRW, PAIRW)]], dst, sem)

    def wait_pair(p, dst, sem):
        pltpu.make_async_copy(
            table_hbm.at[idx_v.at[pl.ds(p * PAIRW, PAIRW)]], dst, sem).wait()

    gather_pair(0, rows0, sem0)

    def group(g, _):
        def quad(qq, _):
            p0 = g * 8 + 2 * qq
            s0 = 4 * qq
            gather_pair(p0 + 1, rows1, sem1)
            wait_pair(p0, rows0, sem0)
            tot_a = product(rows0, 0)
            tot_b = product(rows0, L)

            @pl.when(p0 < NPAIR - 2)
            def _():
                gather_pair(p0 + 2, rows0, sem0)

            wait_pair(p0 + 1, rows1, sem1)
            tot_c = product(rows1, 0)
            tot_d = product(rows1, L)
            tmp_v[pl.ds(s0 * 16, 16)] = tot_a
            tmp_v[pl.ds((s0 + 1) * 16, 16)] = tot_b
            tmp_v[pl.ds((s0 + 2) * 16, 16)] = tot_c
            tmp_v[pl.ds((s0 + 3) * 16, 16)] = tot_d
            return 0

        lax.fori_loop(0, 4, quad, 0)
        acc = jnp.zeros((16,), jnp.float32)
        for j in range(16):
            acc = acc + plsc.load_gather(tmp_v, [iota * 16 + j])
        out_v[pl.ds(g * 16, 16)] = acc
        return 0

    lax.fori_loop(0, GRP, group, 0)
    pltpu.sync_copy(out_v, out_hbm.at[pl.ds(wid * SPW, SPW)])


@jax.jit
def _seg_gps(epsilon, inputs_i32):
    a2 = epsilon.reshape(2, M, L, KK)
    cols = [
        lax.slice(a2, (0, 0, i, i), (2, M, i + 1, i + 32 * 32 + 1),
                  (1, 1, 1, 32))
        for i in range(L)
    ]
    tm = jnp.concatenate(cols, axis=2)
    table = jnp.transpose(tm, (0, 2, 3, 1)).reshape(TROWS, M)

    mesh = plsc.VectorSubcoreMesh(core_axis_name="c", subcore_axis_name="s")
    return pl.kernel(
        _sc_body,
        mesh=mesh,
        compiler_params=pltpu.CompilerParams(
            needs_layout_passes=False, use_tc_tiling_on_sc=False),
        out_type=jax.ShapeDtypeStruct((BATCH,), jnp.float32),
        scratch_types=[
            pltpu.VMEM((SPW, L), jnp.int32),
            pltpu.VMEM((SPW * L,), jnp.int32),
            pltpu.VMEM((PAIRW, M), jnp.float32),
            pltpu.VMEM((PAIRW, M), jnp.float32),
            pltpu.VMEM((256,), jnp.float32),
            pltpu.VMEM((SPW,), jnp.float32),
            pltpu.SemaphoreType.DMA,
            pltpu.SemaphoreType.DMA,
        ],
    )(table, inputs_i32)


def kernel(inputs, epsilon):
    return _seg_gps(epsilon, inputs.astype(jnp.int32))


# speedup vs baseline: 1.8342x; 1.8342x over previous
import functools

import jax
import jax.numpy as jnp
from jax import lax
from jax.experimental import pallas as pl
from jax.experimental.pallas import tpu as pltpu
from jax.experimental.pallas import tpu_sc as plsc

L = 64
M = 64
BATCH = 4096
NUP = 33
KK = NUP * NUP
TROWS = 2 * L * NUP * NUP
S_STRIDE = L * NUP * NUP
I_STRIDE = NUP * NUP + 1
NU_STRIDE = NUP - 1

_NC, _NS = 2, 16
NW = _NC * _NS
SPW = BATCH // NW
GRP = SPW // 16
PAIRW = 2 * L
NPAIR = SPW // 2


def _sc_body(table_hbm, inputs_hbm, out_hbm, in_v, idx_v, rows0, rows1,
             tmp_v, out_v, sem0, sem1):
    sub = lax.axis_index("s")
    core = lax.axis_index("c")
    wid = sub * _NC + core
    iota = lax.iota(jnp.int32, 16)

    pltpu.sync_copy(inputs_hbm.at[pl.ds(wid * SPW, SPW)], in_v)

    def bidx(t, _):
        carry = jnp.float32(0)
        for k in range(4):
            sv = in_v[t, pl.ds(16 * k, 16)]
            sf = sv.astype(jnp.float32)
            incl = jnp.cumsum(sf)
            nu = (incl - sf + carry).astype(jnp.int32)
            carry = carry + jnp.sum(sf)
            idx_v[pl.ds(t * L + 16 * k, 16)] = (
                sv * S_STRIDE + (iota + 16 * k) * I_STRIDE + nu * NU_STRIDE)
        return 0

    lax.fori_loop(0, SPW, bidx, 0)

    def product(rows_v, off):
        def prod(j, accs):
            accs = list(accs)
            for r in range(8):
                row = off + 8 * j + r
                c = (r % 2) * 4
                for k in range(4):
                    accs[c + k] = accs[c + k] * rows_v[row, pl.ds(16 * k, 16)]
            return tuple(accs)

        ones = jnp.ones((16,), jnp.float32)
        accs = lax.fori_loop(0, L // 8, prod, (ones,) * 8)
        return (accs[0] * accs[4] + accs[1] * accs[5]
                + accs[2] * accs[6] + accs[3] * accs[7])

    def gather_pair(p, dst, sem):
        return pltpu.async_copy(
            table_hbm.at[idx_v.at[pl.ds(p * PAIRW, PAIRW)]], dst, sem)

    def wait_pair(p, dst, sem):
        pltpu.make_async_copy(
            table_hbm.at[idx_v.at[pl.ds(p * PAIRW, PAIRW)]], dst, sem).wait()

    gather_pair(0, rows0, sem0)

    def group(g, _):
        def quad(qq, _):
            p0 = g * 8 + 2 * qq
            s0 = 4 * qq
            gather_pair(p0 + 1, rows1, sem1)
            wait_pair(p0, rows0, sem0)
            tot_a = product(rows0, 0)
            tot_b = product(rows0, L)

            @pl.when(p0 < NPAIR - 2)
            def _():
                gather_pair(p0 + 2, rows0, sem0)

            wait_pair(p0 + 1, rows1, sem1)
            tot_c = product(rows1, 0)
            tot_d = product(rows1, L)
            tmp_v[pl.ds(s0 * 16, 16)] = tot_a
            tmp_v[pl.ds((s0 + 1) * 16, 16)] = tot_b
            tmp_v[pl.ds((s0 + 2) * 16, 16)] = tot_c
            tmp_v[pl.ds((s0 + 3) * 16, 16)] = tot_d
            return 0

        lax.fori_loop(0, 4, quad, 0)
        acc = jnp.zeros((16,), jnp.float32)
        for j in range(16):
            acc = acc + plsc.load_gather(tmp_v, [iota * 16 + j])
        out_v[pl.ds(g * 16, 16)] = acc
        return 0

    lax.fori_loop(0, GRP, group, 0)
    pltpu.sync_copy(out_v, out_hbm.at[pl.ds(wid * SPW, SPW)])


@jax.jit
def _seg_gps(epsilon, inputs_i32):
    table = jnp.transpose(epsilon, (0, 2, 3, 4, 1)).reshape(TROWS, M)

    mesh = plsc.VectorSubcoreMesh(core_axis_name="c", subcore_axis_name="s")
    return pl.kernel(
        _sc_body,
        mesh=mesh,
        compiler_params=pltpu.CompilerParams(
            needs_layout_passes=False, use_tc_tiling_on_sc=False),
        out_type=jax.ShapeDtypeStruct((BATCH,), jnp.float32),
        scratch_types=[
            pltpu.VMEM((SPW, L), jnp.int32),
            pltpu.VMEM((SPW * L,), jnp.int32),
            pltpu.VMEM((PAIRW, M), jnp.float32),
            pltpu.VMEM((PAIRW, M), jnp.float32),
            pltpu.VMEM((256,), jnp.float32),
            pltpu.VMEM((SPW,), jnp.float32),
            pltpu.SemaphoreType.DMA,
            pltpu.SemaphoreType.DMA,
        ],
    )(table, inputs_i32)


def kernel(inputs, epsilon):
    return _seg_gps(epsilon, inputs.astype(jnp.int32))


# speedup vs baseline: 1.8449x; 1.0058x over previous
import functools

import jax
import jax.numpy as jnp
from jax import lax
from jax.experimental import pallas as pl
from jax.experimental.pallas import tpu as pltpu
from jax.experimental.pallas import tpu_sc as plsc

L = 64
M = 64
BATCH = 4096
NUP = 33
KK = NUP * NUP
TROWS = 2 * L * NUP * NUP
S_STRIDE = L * NUP * NUP
I_STRIDE = NUP * NUP + 1
NU_STRIDE = NUP - 1

_NC, _NS = 2, 16
NW = _NC * _NS
SPW = BATCH // NW
GRP = SPW // 16
PAIRW = 2 * L
NPAIR = SPW // 2


def _sc_body(table_hbm, inputs_hbm, out_hbm, in_v, idx_v, rows0, rows1,
             tmp_v, out_v, sem0, sem1):
    sub = lax.axis_index("s")
    core = lax.axis_index("c")
    wid = sub * _NC + core
    iota = lax.iota(jnp.int32, 16)

    pltpu.sync_copy(inputs_hbm.at[pl.ds(wid * SPW, SPW)], in_v)

    def bidx(t, _):
        carry = jnp.float32(0)
        for k in range(4):
            sv = in_v[t, pl.ds(16 * k, 16)]
            sf = sv.astype(jnp.float32)
            incl = jnp.cumsum(sf)
            nu = (incl - sf + carry).astype(jnp.int32)
            carry = carry + jnp.sum(sf)
            idx_v[pl.ds(t * L + 16 * k, 16)] = (
                sv * S_STRIDE + (iota + 16 * k) * I_STRIDE + nu * NU_STRIDE)
        return 0

    lax.fori_loop(0, SPW, bidx, 0)

    def product(rows_v, off):
        def prod(j, accs):
            accs = list(accs)
            for r in range(8):
                row = off + 8 * j + r
                c = (r % 2) * 4
                for k in range(4):
                    accs[c + k] = accs[c + k] * rows_v[row, pl.ds(16 * k, 16)]
            return tuple(accs)

        ones = jnp.ones((16,), jnp.float32)
        accs = lax.fori_loop(0, L // 8, prod, (ones,) * 8)
        return (accs[0] * accs[4] + accs[1] * accs[5]
                + accs[2] * accs[6] + accs[3] * accs[7])

    def gather_pair(p, dst, sem):
        return pltpu.async_copy(
            table_hbm.at[idx_v.at[pl.ds(p * PAIRW, PAIRW)]], dst, sem)

    def wait_pair(p, dst, sem):
        pltpu.make_async_copy(
            table_hbm.at[idx_v.at[pl.ds(p * PAIRW, PAIRW)]], dst, sem).wait()

    gather_pair(0, rows0, sem0)

    def group(g, _):
        def quad(qq, _):
            p0 = g * 8 + 2 * qq
            s0 = 4 * qq
            gather_pair(p0 + 1, rows1, sem1)
            wait_pair(p0, rows0, sem0)
            tot_a = product(rows0, 0)
            tot_b = product(rows0, L)

            @pl.when(p0 < NPAIR - 2)
            def _():
                gather_pair(p0 + 2, rows0, sem0)

            wait_pair(p0 + 1, rows1, sem1)
            tot_c = product(rows1, 0)
            tot_d = product(rows1, L)
            tmp_v[pl.ds(s0 * 16, 16)] = tot_a
            tmp_v[pl.ds((s0 + 1) * 16, 16)] = tot_b
            tmp_v[pl.ds((s0 + 2) * 16, 16)] = tot_c
            tmp_v[pl.ds((s0 + 3) * 16, 16)] = tot_d
            return 0

        lax.fori_loop(0, 4, quad, 0)
        acc = jnp.zeros((16,), jnp.float32)
        for j in range(16):
            acc = acc + plsc.load_gather(tmp_v, [iota * 16 + j])
        out_v[pl.ds(g * 16, 16)] = acc
        return 0

    lax.fori_loop(0, GRP, group, 0)
    pltpu.sync_copy(out_v, out_hbm.at[pl.ds(wid * SPW, SPW)])


@jax.jit
def _seg_gps(epsilon, inputs_i32):
    table = lax.reshape(epsilon, (TROWS, M), dimensions=(0, 2, 3, 4, 1))

    mesh = plsc.VectorSubcoreMesh(core_axis_name="c", subcore_axis_name="s")
    return pl.kernel(
        _sc_body,
        mesh=mesh,
        compiler_params=pltpu.CompilerParams(
            needs_layout_passes=False, use_tc_tiling_on_sc=False),
        out_type=jax.ShapeDtypeStruct((BATCH,), jnp.float32),
        scratch_types=[
            pltpu.VMEM((SPW, L), jnp.int32),
            pltpu.VMEM((SPW * L,), jnp.int32),
            pltpu.VMEM((PAIRW, M), jnp.float32),
            pltpu.VMEM((PAIRW, M), jnp.float32),
            pltpu.VMEM((256,), jnp.float32),
            pltpu.VMEM((SPW,), jnp.float32),
            pltpu.SemaphoreType.DMA,
            pltpu.SemaphoreType.DMA,
        ],
    )(table, inputs_i32)


def kernel(inputs, epsilon):
    return _seg_gps(epsilon, inputs.astype(jnp.int32))
